# baseline (speedup 1.0000x reference)
import functools

import jax
import jax.numpy as jnp
from jax import lax
from jax.experimental import pallas as pl
from jax.experimental.pallas import tpu as pltpu
from jax.experimental.pallas import tpu_sc as plsc

B = 16384
D = 32
L = 16
NW = 32
C = B // NW
G = C // L
MARGIN = 1.0
EPS = 1e-6

_mesh = plsc.VectorSubcoreMesh(core_axis_name="c", subcore_axis_name="s")


def _sc_body(ent_hbm, rel_hbm, idx_hbm, out_hbm,
             iph, ipt, ipr, inh, int_, inr,
             ph, pt, pr, nh, nt, nr,
             sqp, sqn, sem):
    cid = lax.axis_index("c")
    sid = lax.axis_index("s")
    wid = sid * 2 + cid
    base = wid * C

    pltpu.sync_copy(idx_hbm.at[pl.ds(0 * B + base, C)], iph)
    pltpu.sync_copy(idx_hbm.at[pl.ds(1 * B + base, C)], ipt)
    pltpu.sync_copy(idx_hbm.at[pl.ds(2 * B + base, C)], ipr)
    pltpu.sync_copy(idx_hbm.at[pl.ds(3 * B + base, C)], inh)
    pltpu.sync_copy(idx_hbm.at[pl.ds(4 * B + base, C)], int_)
    pltpu.sync_copy(idx_hbm.at[pl.ds(5 * B + base, C)], inr)

    cps = [
        pltpu.async_copy(ent_hbm.at[iph], ph, sem),
        pltpu.async_copy(ent_hbm.at[ipt], pt, sem),
        pltpu.async_copy(rel_hbm.at[ipr], pr, sem),
        pltpu.async_copy(ent_hbm.at[inh], nh, sem),
        pltpu.async_copy(ent_hbm.at[int_], nt, sem),
        pltpu.async_copy(rel_hbm.at[inr], nr, sem),
    ]
    for cp in cps:
        cp.wait()

    def row_sq(hbuf, rbuf, tbuf, rix):
        h0 = hbuf[rix, pl.ds(0, L)]
        h1 = hbuf[rix, pl.ds(L, L)]
        r0 = rbuf[rix, pl.ds(0, L)]
        r1 = rbuf[rix, pl.ds(L, L)]
        t0 = tbuf[rix, pl.ds(0, L)]
        t1 = tbuf[rix, pl.ds(L, L)]
        d0 = h0 + r0 - t0 + EPS
        d1 = h1 + r1 - t1 + EPS
        return d0 * d0 + d1 * d1

    def group_body(g, carry):
        for k in range(L):
            r = g * L + k
            sqp[pl.ds(r * L, L)] = row_sq(ph, pr, pt, r)
            sqn[pl.ds(r * L, L)] = row_sq(nh, nr, nt, r)
        return carry

    lax.fori_loop(0, G, group_body, 0)

    pltpu.sync_copy(sqp, out_hbm.at[pl.ds(base * L, C * L)])
    pltpu.sync_copy(sqn, out_hbm.at[pl.ds(B * L + base * L, C * L)])


_sc_distances = functools.partial(
    pl.kernel,
    out_type=jax.ShapeDtypeStruct((2 * B * L,), jnp.float32),
    mesh=_mesh,
    compiler_params=pltpu.CompilerParams(use_tc_tiling_on_sc=False),
    scratch_types=[
        pltpu.VMEM((C,), jnp.int32),
        pltpu.VMEM((C,), jnp.int32),
        pltpu.VMEM((C,), jnp.int32),
        pltpu.VMEM((C,), jnp.int32),
        pltpu.VMEM((C,), jnp.int32),
        pltpu.VMEM((C,), jnp.int32),
        pltpu.VMEM((C, D), jnp.float32),
        pltpu.VMEM((C, D), jnp.float32),
        pltpu.VMEM((C, D), jnp.float32),
        pltpu.VMEM((C, D), jnp.float32),
        pltpu.VMEM((C, D), jnp.float32),
        pltpu.VMEM((C, D), jnp.float32),
        pltpu.VMEM((C * L,), jnp.float32),
        pltpu.VMEM((C * L,), jnp.float32),
        pltpu.SemaphoreType.DMA,
    ],
)(_sc_body)

_ROWS = 2 * B * L // 128
_HALF = _ROWS // 2


def _finish_body(x_ref, o_ref):
    x = x_ref[...]
    i128 = lax.broadcasted_iota(jnp.int32, (128, 8), 0)
    i8 = lax.broadcasted_iota(jnp.int32, (128, 8), 1)
    s_mat = jnp.where(i128 // L == i8, 1.0, 0.0).astype(jnp.float32)
    d2p = jnp.dot(x[:_HALF], s_mat, preferred_element_type=jnp.float32)
    d2n = jnp.dot(x[_HALF:], s_mat, preferred_element_type=jnp.float32)
    m = jnp.sqrt(d2p) - jnp.sqrt(d2n) + MARGIN
    o_ref[...] = jnp.sum(jnp.maximum(m, 0.0), keepdims=True) * (1.0 / B)


_finish = pl.pallas_call(
    _finish_body,
    out_shape=jax.ShapeDtypeStruct((1, 1), jnp.float32),
)


def kernel(pos_x, neg_x, entity_weight, relation_weight):
    pos = pos_x.astype(jnp.int32)
    neg = neg_x.astype(jnp.int32)
    idx_flat = jnp.concatenate([
        pos[:, 0], pos[:, 2], pos[:, 1],
        neg[:, 0], neg[:, 2], neg[:, 1],
    ])
    sq = _sc_distances(entity_weight, relation_weight, idx_flat)
    return _finish(sq.reshape(_ROWS, 128))[0, 0]
